# baseline (speedup 1.0000x reference)
import functools

import jax
import jax.numpy as jnp
from jax import lax
from jax.experimental import pallas as pl
from jax.experimental.pallas import tpu as pltpu
from jax.experimental.pallas import tpu_sc as plsc

N, C, D, H, W = 2, 1, 128, 128, 128
P = D * H * W
NP = N * P
NW = 32
PER_TILE = NP // NW
K = 2048
NCH = PER_TILE // K
ROWS = K // 128
OFFS = [0, 1, W, W + 1, H * W, H * W + 1, H * W + W, H * W + W + 1]


def _sc_warp(img, flw, grd):
    mesh = plsc.VectorSubcoreMesh(core_axis_name="c", subcore_axis_name="s")

    @functools.partial(
        pl.kernel,
        out_type=jax.ShapeDtypeStruct((NP,), jnp.float32),
        mesh=mesh,
        compiler_params=pltpu.CompilerParams(needs_layout_passes=False),
        scratch_types=[
            pltpu.VMEM((3 * K,), jnp.float32),
            pltpu.VMEM((3 * K,), jnp.float32),
            pltpu.VMEM((3 * K,), jnp.int32),
            pltpu.VMEM((3 * K,), jnp.float32),
            pltpu.VMEM((8 * ROWS, 128), jnp.int32),
            pltpu.VMEM((8 * ROWS, 128), jnp.float32),
            pltpu.VMEM((K,), jnp.float32),
            pltpu.SemaphoreType.DMA,
        ],
    )
    def warp(img_h, flw_h, grd_h, out_h, gbuf, fbuf, tibuf, frbuf,
             idxbuf, vbuf, outbuf, gsem):
        wid = lax.axis_index("s") * 2 + lax.axis_index("c")
        lane = lax.iota(jnp.int32, 16)

        @pl.loop(0, NCH)
        def _chunk(ch):
            p0 = pl.multiple_of(wid * PER_TILE + ch * K, K)
            n_off = (p0 // P) * P

            pltpu.sync_copy(grd_h.at[pl.ds(3 * p0, 3 * K)], gbuf)
            pltpu.sync_copy(flw_h.at[pl.ds(3 * p0, 3 * K)], fbuf)

            @pl.loop(0, 3 * K // 16)
            def _ew(i):
                o = pl.multiple_of(i * 16, 16)
                s = gbuf[pl.ds(o, 16)] + fbuf[pl.ds(o, 16)]
                t = (s + 1.0) * 0.5 * 127.0
                t = jnp.clip(t, 0.0, 127.0)
                ti = t.astype(jnp.int32)
                tibuf[pl.ds(o, 16)] = ti
                frbuf[pl.ds(o, 16)] = t - ti.astype(jnp.float32)

            @pl.loop(0, K // 16)
            def _idx(j):
                b3 = j * 48 + 3 * lane
                xi = plsc.load_gather(tibuf, [b3])
                yi = plsc.load_gather(tibuf, [b3 + 1])
                zi = plsc.load_gather(tibuf, [b3 + 2])
                base = n_off + (zi << 14) + (yi << 7) + xi
                r8 = j >> 3
                co = (j & 7) * 16
                for c in range(8):
                    iv = jnp.minimum(base + OFFS[c], NP - 1)
                    idxbuf[c * ROWS + r8, pl.ds(co, 16)] = iv

            @pl.loop(0, 8 * ROWS)
            def _fire(r):
                pltpu.async_copy(img_h.at[idxbuf.at[r]], vbuf.at[r], gsem)

            @pl.loop(0, 8 * ROWS)
            def _drain(r):
                pltpu.make_async_copy(img_h.at[idxbuf.at[r]], vbuf.at[r],
                                      gsem).wait()

            @pl.loop(0, K // 16)
            def _acc(j):
                b3 = j * 48 + 3 * lane
                wx = plsc.load_gather(frbuf, [b3])
                wy = plsc.load_gather(frbuf, [b3 + 1])
                wz = plsc.load_gather(frbuf, [b3 + 2])
                ux = 1.0 - wx
                uy = 1.0 - wy
                uz = 1.0 - wz
                r8 = j >> 3
                co = (j & 7) * 16
                a = uz * uy
                b = uz * wy
                cc = wz * uy
                dd = wz * wy
                ws = (a * ux, a * wx, b * ux, b * wx,
                      cc * ux, cc * wx, dd * ux, dd * wx)
                acc = vbuf[0 * ROWS + r8, pl.ds(co, 16)] * ws[0]
                for c in range(1, 8):
                    acc = acc + vbuf[c * ROWS + r8, pl.ds(co, 16)] * ws[c]
                outbuf[pl.ds(j * 16, 16)] = acc

            pltpu.sync_copy(outbuf, out_h.at[pl.ds(p0, K)])

    return warp(img, flw, grd)


def kernel(input_image, flow, grid):
    assert input_image.shape == (N, C, D, H, W)
    out = _sc_warp(input_image.reshape(-1), flow.reshape(-1),
                   grid.reshape(-1))
    return out.reshape(input_image.shape)


# speedup vs baseline: 1.5455x; 1.5455x over previous
import functools

import jax
import jax.numpy as jnp
from jax import lax
from jax.experimental import pallas as pl
from jax.experimental.pallas import tpu as pltpu
from jax.experimental.pallas import tpu_sc as plsc

N, C, D, H, W = 2, 1, 128, 128, 128
P = D * H * W
NP = N * P
HW = H * W
NW = 32
PER_TILE = NP // NW
K = 2048
NCH = PER_TILE // K
ROWS = K // 128
OFFS = [0, 1, W, W + 1, H * W, H * W + 1, H * W + W, H * W + W + 1]


def _sc_warp(img, flw, grd):
    mesh = plsc.VectorSubcoreMesh(core_axis_name="c", subcore_axis_name="s")

    @functools.partial(
        pl.kernel,
        out_type=jax.ShapeDtypeStruct((NP,), jnp.float32),
        mesh=mesh,
        compiler_params=pltpu.CompilerParams(needs_layout_passes=False),
        scratch_types=[
            pltpu.VMEM((3 * K,), jnp.float32),
            pltpu.VMEM((3 * K,), jnp.float32),
            pltpu.VMEM((3 * K,), jnp.float32),
            pltpu.VMEM((8 * ROWS, 128), jnp.int32),
            pltpu.VMEM((8 * ROWS, 128), jnp.float32),
            pltpu.VMEM((K,), jnp.float32),
            pltpu.SemaphoreType.DMA,
        ],
    )
    def warp(img_h, flw_h, grd_h, out_h, gbuf, fbuf, frbuf,
             idxbuf, vbuf, outbuf, gsem):
        wid = lax.axis_index("s") * 2 + lax.axis_index("c")

        @pl.loop(0, NCH)
        def _chunk(ch):
            p0 = pl.multiple_of(wid * PER_TILE + ch * K, K)
            n_off = (p0 // P) * P
            nz = p0 // HW
            yo = p0 - nz * HW
            for c in range(3):
                src = pl.multiple_of((nz * 3 + c) * HW + yo, K)
                pltpu.sync_copy(grd_h.at[pl.ds(src, K)],
                                gbuf.at[pl.ds(c * K, K)])
                pltpu.sync_copy(flw_h.at[pl.ds(src, K)],
                                fbuf.at[pl.ds(c * K, K)])

            @pl.loop(0, K // 16)
            def _idx(j):
                o = pl.multiple_of(j * 16, 16)
                comp = []
                for c in range(3):
                    s = gbuf[pl.ds(c * K + o, 16)] + fbuf[pl.ds(c * K + o, 16)]
                    t = (s + 1.0) * 0.5 * 127.0
                    t = jnp.clip(t, 0.0, 127.0)
                    ti = t.astype(jnp.int32)
                    frbuf[pl.ds(c * K + o, 16)] = t - ti.astype(jnp.float32)
                    comp.append(ti)
                base = n_off + (comp[2] << 14) + (comp[1] << 7) + comp[0]
                r8 = j >> 3
                co = (j & 7) * 16
                for c in range(8):
                    iv = jnp.minimum(base + OFFS[c], NP - 1)
                    idxbuf[c * ROWS + r8, pl.ds(co, 16)] = iv

            @pl.loop(0, 8 * ROWS)
            def _fire(r):
                pltpu.async_copy(img_h.at[idxbuf.at[r]], vbuf.at[r], gsem)

            @pl.loop(0, 8 * ROWS)
            def _drain(r):
                pltpu.make_async_copy(img_h.at[idxbuf.at[r]], vbuf.at[r],
                                      gsem).wait()

            @pl.loop(0, K // 16)
            def _acc(j):
                o = pl.multiple_of(j * 16, 16)
                wx = frbuf[pl.ds(0 * K + o, 16)]
                wy = frbuf[pl.ds(1 * K + o, 16)]
                wz = frbuf[pl.ds(2 * K + o, 16)]
                ux = 1.0 - wx
                uy = 1.0 - wy
                uz = 1.0 - wz
                r8 = j >> 3
                co = (j & 7) * 16
                a = uz * uy
                b = uz * wy
                cc = wz * uy
                dd = wz * wy
                ws = (a * ux, a * wx, b * ux, b * wx,
                      cc * ux, cc * wx, dd * ux, dd * wx)
                acc = vbuf[0 * ROWS + r8, pl.ds(co, 16)] * ws[0]
                for c in range(1, 8):
                    acc = acc + vbuf[c * ROWS + r8, pl.ds(co, 16)] * ws[c]
                outbuf[pl.ds(o, 16)] = acc

            pltpu.sync_copy(outbuf, out_h.at[pl.ds(p0, K)])

    return warp(img, flw, grd)


def kernel(input_image, flow, grid):
    assert input_image.shape == (N, C, D, H, W)
    ft = flow.transpose(0, 1, 4, 2, 3).reshape(-1)
    gt = grid.transpose(0, 1, 4, 2, 3).reshape(-1)
    out = _sc_warp(input_image.reshape(-1), ft, gt)
    return out.reshape(input_image.shape)


# speedup vs baseline: 22.6825x; 14.6768x over previous
import functools

import jax
import jax.numpy as jnp
from jax import lax
from jax.experimental import pallas as pl
from jax.experimental.pallas import tpu as pltpu
from jax.experimental.pallas import tpu_sc as plsc

N, C, D, H, W = 2, 1, 128, 128, 128
P = D * H * W
NP = N * P
HW = H * W
NW = 32
PER_TILE = NP // NW
K = 2048
NCH = PER_TILE // K
GRP = K // 128
NPLANES = N * D
TPT = NPLANES // NW

_CPARAMS = pltpu.CompilerParams(
    needs_layout_passes=False, use_tc_tiling_on_sc=False)


def _sc_build(img):
    mesh = plsc.VectorSubcoreMesh(core_axis_name="c", subcore_axis_name="s")

    @functools.partial(
        pl.kernel,
        out_type=jax.ShapeDtypeStruct((NP, 8), jnp.float32),
        mesh=mesh,
        compiler_params=_CPARAMS,
        scratch_types=[
            pltpu.VMEM((HW,), jnp.float32),
            pltpu.VMEM((HW,), jnp.float32),
            pltpu.VMEM((2, 2048, 8), jnp.float32),
            pltpu.SemaphoreType.DMA,
        ],
    )
    def build(img_h, tab_h, pbuf0, pbuf1, obuf, osem):
        wid = lax.axis_index("s") * 2 + lax.axis_index("c")
        lane = lax.iota(jnp.int32, 16)
        cv0 = lane * 0

        @pl.loop(0, TPT)
        def _task(t):
            q = wid * TPT + t
            zoff = pl.multiple_of(q * HW, HW)
            z = q & (D - 1)
            zp_off = pl.multiple_of(
                jnp.where(z == D - 1, zoff, zoff + HW), HW)
            pltpu.sync_copy(img_h.at[pl.ds(zoff, HW)], pbuf0)
            pltpu.sync_copy(img_h.at[pl.ds(zp_off, HW)], pbuf1)

            @pl.loop(0, 8)
            def _ychunk(yc):
                buf = yc & 1
                dst = pl.multiple_of(q * HW + yc * 2048, 2048)

                @pl.when(yc >= 2)
                def _():
                    pltpu.make_async_copy(
                        obuf.at[buf], tab_h.at[pl.ds(dst, 2048)],
                        osem).wait()

                @pl.loop(0, 16)
                def _line(l):
                    y = yc * 16 + l
                    ro0 = y * W
                    ro1 = jnp.minimum(y + 1, H - 1) * W
                    ob = obuf.at[buf]

                    @pl.loop(0, 8)
                    def _xg(xg):
                        xo = xg * 16
                        xe = xo + lane
                        xc = jnp.minimum(xe + 1, W - 1)
                        pos8 = (l * W + xe) * 8
                        c = 0
                        for pb in (pbuf0, pbuf1):
                            for ro in (ro0, ro1):
                                ve = pb[pl.ds(pl.multiple_of(ro + xo, 16),
                                              16)]
                                vo = plsc.load_gather(pb, [ro + xc])
                                pt = l * W + xe
                                plsc.store_scatter(ob, [pt, cv0 + c], ve)
                                plsc.store_scatter(ob, [pt, cv0 + c + 1], vo)
                                c += 2

                pltpu.async_copy(obuf.at[buf],
                                 tab_h.at[pl.ds(dst, 2048)], osem)

            @pl.loop(0, 2)
            def _tail(i):
                pltpu.make_async_copy(
                    obuf.at[i], tab_h.at[pl.ds(0, 2048)],
                    osem).wait()

    return build(img)


def _sc_warp(tab, flw, grd):
    mesh = plsc.VectorSubcoreMesh(core_axis_name="c", subcore_axis_name="s")

    @functools.partial(
        pl.kernel,
        out_type=jax.ShapeDtypeStruct((NP,), jnp.float32),
        mesh=mesh,
        compiler_params=_CPARAMS,
        scratch_types=[
            pltpu.VMEM((3 * K,), jnp.float32),
            pltpu.VMEM((3 * K,), jnp.float32),
            pltpu.VMEM((3 * K,), jnp.float32),
            pltpu.VMEM((GRP, 128), jnp.int32),
            pltpu.VMEM((K, 8), jnp.float32),
            pltpu.VMEM((K,), jnp.float32),
            pltpu.SemaphoreType.DMA,
        ],
    )
    def warp(tab_h, flw_h, grd_h, out_h, gbuf, fbuf, frbuf,
             idxbuf, vbuf, outbuf, gsem):
        wid = lax.axis_index("s") * 2 + lax.axis_index("c")
        tab2 = tab_h
        lane = lax.iota(jnp.int32, 16)
        cvecs = [lane * 0 + c for c in range(8)]

        @pl.loop(0, NCH)
        def _chunk(ch):
            p0 = pl.multiple_of(wid * PER_TILE + ch * K, K)
            n_off = (p0 // P) * P
            nz = p0 // HW
            yo = p0 - nz * HW
            for c in range(3):
                src = pl.multiple_of((nz * 3 + c) * HW + yo, K)
                pltpu.sync_copy(grd_h.at[pl.ds(src, K)],
                                gbuf.at[pl.ds(c * K, K)])
                pltpu.sync_copy(flw_h.at[pl.ds(src, K)],
                                fbuf.at[pl.ds(c * K, K)])

            @pl.loop(0, K // 16)
            def _idx(j):
                o = pl.multiple_of(j * 16, 16)
                comp = []
                for c in range(3):
                    s = gbuf[pl.ds(c * K + o, 16)] + fbuf[pl.ds(c * K + o, 16)]
                    t = (s + 1.0) * 0.5 * 127.0
                    t = jnp.clip(t, 0.0, 127.0)
                    ti = t.astype(jnp.int32)
                    frbuf[pl.ds(c * K + o, 16)] = t - ti.astype(jnp.float32)
                    comp.append(ti)
                base = n_off + (comp[2] << 14) + (comp[1] << 7) + comp[0]
                idxbuf[j >> 3, pl.ds((j & 7) * 16, 16)] = base

            @pl.loop(0, GRP)
            def _fire(g):
                pltpu.async_copy(tab2.at[idxbuf.at[g]],
                                 vbuf.at[pl.ds(g * 128, 128)], gsem)

            @pl.loop(0, GRP)
            def _drain(g):
                pltpu.make_async_copy(tab2.at[idxbuf.at[g]],
                                      vbuf.at[pl.ds(g * 128, 128)],
                                      gsem).wait()

            @pl.loop(0, K // 16)
            def _acc(j):
                o = pl.multiple_of(j * 16, 16)
                wx = frbuf[pl.ds(0 * K + o, 16)]
                wy = frbuf[pl.ds(1 * K + o, 16)]
                wz = frbuf[pl.ds(2 * K + o, 16)]
                ux = 1.0 - wx
                uy = 1.0 - wy
                uz = 1.0 - wz
                a = uz * uy
                b = uz * wy
                cc = wz * uy
                dd = wz * wy
                ws = (a * ux, a * wx, b * ux, b * wx,
                      cc * ux, cc * wx, dd * ux, dd * wx)
                rows = o + lane
                acc = plsc.load_gather(vbuf, [rows, cvecs[0]]) * ws[0]
                for c in range(1, 8):
                    acc = acc + plsc.load_gather(vbuf, [rows, cvecs[c]]) * ws[c]
                outbuf[pl.ds(o, 16)] = acc

            pltpu.sync_copy(outbuf, out_h.at[pl.ds(p0, K)])

    return warp(tab, flw, grd)


def kernel(input_image, flow, grid):
    assert input_image.shape == (N, C, D, H, W)
    ft = flow.transpose(0, 1, 4, 2, 3).reshape(-1)
    gt = grid.transpose(0, 1, 4, 2, 3).reshape(-1)
    tab = _sc_build(input_image.reshape(-1))
    out = _sc_warp(tab, ft, gt)
    return out.reshape(input_image.shape)


# speedup vs baseline: 36.2204x; 1.5968x over previous
import functools

import jax
import jax.numpy as jnp
from jax import lax
from jax.experimental import pallas as pl
from jax.experimental.pallas import tpu as pltpu
from jax.experimental.pallas import tpu_sc as plsc

N, C, D, H, W = 2, 1, 128, 128, 128
P = D * H * W
NP = N * P
HW = H * W
NW = 32
PER_TILE = NP // NW
K = 2048
NCH = PER_TILE // K
GRP = K // 128
NPLANES = N * D
TPT = NPLANES // NW

_CPARAMS = pltpu.CompilerParams(
    needs_layout_passes=False, use_tc_tiling_on_sc=False)


def _sc_build(img):
    mesh = plsc.VectorSubcoreMesh(core_axis_name="c", subcore_axis_name="s")

    @functools.partial(
        pl.kernel,
        out_type=jax.ShapeDtypeStruct((NP, 8), jnp.float32),
        mesh=mesh,
        compiler_params=_CPARAMS,
        scratch_types=[
            pltpu.VMEM((HW,), jnp.float32),
            pltpu.VMEM((HW,), jnp.float32),
            pltpu.VMEM((2, 2048, 8), jnp.float32),
            pltpu.SemaphoreType.DMA,
        ],
    )
    def build(img_h, tab_h, pbuf0, pbuf1, obuf, osem):
        wid = lax.axis_index("s") * 2 + lax.axis_index("c")
        lane = lax.iota(jnp.int32, 16)
        cv0 = lane * 0

        @pl.loop(0, TPT)
        def _task(t):
            q = wid * TPT + t
            zoff = pl.multiple_of(q * HW, HW)
            z = q & (D - 1)
            zp_off = pl.multiple_of(
                jnp.where(z == D - 1, zoff, zoff + HW), HW)
            pltpu.sync_copy(img_h.at[pl.ds(zoff, HW)], pbuf0)
            pltpu.sync_copy(img_h.at[pl.ds(zp_off, HW)], pbuf1)

            @pl.loop(0, 8)
            def _ychunk(yc):
                buf = yc & 1
                dst = pl.multiple_of(q * HW + yc * 2048, 2048)

                @pl.when(yc >= 2)
                def _():
                    pltpu.make_async_copy(
                        obuf.at[buf], tab_h.at[pl.ds(dst, 2048)],
                        osem).wait()

                @pl.loop(0, 16)
                def _line(l):
                    y = yc * 16 + l
                    ro0 = y * W
                    ro1 = jnp.minimum(y + 1, H - 1) * W
                    ob = obuf.at[buf]

                    @pl.loop(0, 8)
                    def _xg(xg):
                        xo = xg * 16
                        xe = xo + lane
                        xc = jnp.minimum(xe + 1, W - 1)
                        pos8 = (l * W + xe) * 8
                        c = 0
                        for pb in (pbuf0, pbuf1):
                            for ro in (ro0, ro1):
                                ve = pb[pl.ds(pl.multiple_of(ro + xo, 16),
                                              16)]
                                vo = plsc.load_gather(pb, [ro + xc])
                                pt = l * W + xe
                                plsc.store_scatter(ob, [pt, cv0 + c], ve)
                                plsc.store_scatter(ob, [pt, cv0 + c + 1], vo)
                                c += 2

                pltpu.async_copy(obuf.at[buf],
                                 tab_h.at[pl.ds(dst, 2048)], osem)

            @pl.loop(0, 2)
            def _tail(i):
                pltpu.make_async_copy(
                    obuf.at[i], tab_h.at[pl.ds(0, 2048)],
                    osem).wait()

    return build(img)


def _sc_warp(tab, flw, grd):
    mesh = plsc.VectorSubcoreMesh(core_axis_name="c", subcore_axis_name="s")

    @functools.partial(
        pl.kernel,
        out_type=jax.ShapeDtypeStruct((NP,), jnp.float32),
        mesh=mesh,
        compiler_params=_CPARAMS,
        scratch_types=[
            pltpu.VMEM((2, 3 * K), jnp.float32),
            pltpu.VMEM((2, 3 * K), jnp.float32),
            pltpu.VMEM((2, 3 * K), jnp.float32),
            pltpu.VMEM((2, GRP, 128), jnp.int32),
            pltpu.VMEM((2, K, 8), jnp.float32),
            pltpu.VMEM((2, K), jnp.float32),
            pltpu.SemaphoreType.DMA,
            pltpu.SemaphoreType.DMA,
            pltpu.SemaphoreType.DMA,
        ],
    )
    def warp(tab_h, flw_h, grd_h, out_h, gbuf, fbuf, frbuf,
             idxbuf, vbuf, outbuf, isem, gsem, osem):
        wid = lax.axis_index("s") * 2 + lax.axis_index("c")
        tab2 = tab_h
        lane = lax.iota(jnp.int32, 16)
        cvecs = [lane * 0 + c for c in range(8)]

        def in_copies(ch, slot):
            p0 = pl.multiple_of(wid * PER_TILE + ch * K, K)
            nz = p0 // HW
            yo = p0 - nz * HW
            for c in range(3):
                src = pl.multiple_of((nz * 3 + c) * HW + yo, K)
                yield (grd_h.at[pl.ds(src, K)],
                       gbuf.at[slot, pl.ds(c * K, K)])
                yield (flw_h.at[pl.ds(src, K)],
                       fbuf.at[slot, pl.ds(c * K, K)])

        def fire_in(ch, slot):
            for s, d in in_copies(ch, slot):
                pltpu.async_copy(s, d, isem)

        def wait_in(ch, slot):
            for s, d in in_copies(ch, slot):
                pltpu.make_async_copy(s, d, isem).wait()

        def gather_copies(slot):
            for g in range(GRP):
                yield (tab2.at[idxbuf.at[slot, g]],
                       vbuf.at[slot, pl.ds(g * 128, 128)])

        def compute_idx(ch, slot):
            p0 = pl.multiple_of(wid * PER_TILE + ch * K, K)
            n_off = (p0 // P) * P

            @pl.loop(0, K // 16)
            def _idx(j):
                o = pl.multiple_of(j * 16, 16)
                comp = []
                for c in range(3):
                    s = (gbuf[slot, pl.ds(c * K + o, 16)]
                         + fbuf[slot, pl.ds(c * K + o, 16)])
                    t = (s + 1.0) * 0.5 * 127.0
                    t = jnp.clip(t, 0.0, 127.0)
                    ti = t.astype(jnp.int32)
                    frbuf[slot, pl.ds(c * K + o, 16)] = (
                        t - ti.astype(jnp.float32))
                    comp.append(ti)
                base = n_off + (comp[2] << 14) + (comp[1] << 7) + comp[0]
                idxbuf[slot, j >> 3, pl.ds((j & 7) * 16, 16)] = base

        def blend(ch, slot):
            p0 = pl.multiple_of(wid * PER_TILE + ch * K, K)

            @pl.loop(0, K // 16)
            def _acc(j):
                o = pl.multiple_of(j * 16, 16)
                wx = frbuf[slot, pl.ds(0 * K + o, 16)]
                wy = frbuf[slot, pl.ds(1 * K + o, 16)]
                wz = frbuf[slot, pl.ds(2 * K + o, 16)]
                ux = 1.0 - wx
                uy = 1.0 - wy
                uz = 1.0 - wz
                a = uz * uy
                b = uz * wy
                cc = wz * uy
                dd = wz * wy
                ws = (a * ux, a * wx, b * ux, b * wx,
                      cc * ux, cc * wx, dd * ux, dd * wx)
                rows = o + lane
                vb = vbuf.at[slot]
                acc = plsc.load_gather(vb, [rows, cvecs[0]]) * ws[0]
                for c in range(1, 8):
                    acc = acc + plsc.load_gather(vb, [rows, cvecs[c]]) * ws[c]
                outbuf[slot, pl.ds(o, 16)] = acc

        def out_copy(ch, slot):
            p0 = pl.multiple_of(wid * PER_TILE + ch * K, K)
            return (outbuf.at[slot], out_h.at[pl.ds(p0, K)])

        fire_in(0, 0)

        @pl.loop(0, NCH)
        def _chunk(ch):
            slot = ch & 1
            pslot = 1 - slot

            @pl.when(ch + 1 < NCH)
            def _():
                fire_in(ch + 1, pslot)

            wait_in(ch, slot)
            compute_idx(ch, slot)

            @pl.when(ch >= 1)
            def _():
                for s, d in gather_copies(pslot):
                    pltpu.make_async_copy(s, d, gsem).wait()

            for s, d in gather_copies(slot):
                pltpu.async_copy(s, d, gsem)

            @pl.when(ch >= 1)
            def _():
                @pl.when(ch >= 3)
                def _():
                    s, d = out_copy(ch - 3, pslot)
                    pltpu.make_async_copy(s, d, osem).wait()

                blend(ch - 1, pslot)
                s, d = out_copy(ch - 1, pslot)
                pltpu.async_copy(s, d, osem)

        last = NCH - 1
        lslot = last & 1
        for s, d in gather_copies(lslot):
            pltpu.make_async_copy(s, d, gsem).wait()
        s, d = out_copy(last - 2, lslot)
        pltpu.make_async_copy(s, d, osem).wait()
        blend(last, lslot)
        s, d = out_copy(last - 1, 1 - lslot)
        pltpu.make_async_copy(s, d, osem).wait()
        s, d = out_copy(last, lslot)
        pltpu.sync_copy(s, d)

    return warp(tab, flw, grd)


def kernel(input_image, flow, grid):
    assert input_image.shape == (N, C, D, H, W)
    ft = flow.transpose(0, 1, 4, 2, 3).reshape(-1)
    gt = grid.transpose(0, 1, 4, 2, 3).reshape(-1)
    tab = _sc_build(input_image.reshape(-1))
    out = _sc_warp(tab, ft, gt)
    return out.reshape(input_image.shape)


# speedup vs baseline: 39.6454x; 1.0946x over previous
import functools

import jax
import jax.numpy as jnp
from jax import lax
from jax.experimental import pallas as pl
from jax.experimental.pallas import tpu as pltpu
from jax.experimental.pallas import tpu_sc as plsc

N, C, D, H, W = 2, 1, 128, 128, 128
P = D * H * W
NP = N * P
HW = H * W
NW = 32
PER_TILE = NP // NW
K = 2048
NCH = PER_TILE // K
NPLANES = N * D
TPT = NPLANES // NW

_CPARAMS = pltpu.CompilerParams(
    needs_layout_passes=False, use_tc_tiling_on_sc=False)


def _tc_coords(flw5, grd5):
    def body(f_ref, g_ref, fr_ref, b_ref):
        n = pl.program_id(0)
        s = f_ref[0, 0] + g_ref[0, 0]
        t = (s + 1.0) * 0.5 * 127.0
        t = jnp.clip(t, 0.0, 127.0)
        ti = t.astype(jnp.int32)
        fr_ref[0, 0] = t - ti.astype(jnp.float32)
        b_ref[0, 0] = (ti[0] + (ti[1] << 7) + (ti[2] << 14)) + n * P

    blk5 = pl.BlockSpec((1, 1, 3, H, W), lambda n, z: (n, z, 0, 0, 0))
    blk4 = pl.BlockSpec((1, 1, H, W), lambda n, z: (n, z, 0, 0))
    return pl.pallas_call(
        body,
        grid=(N, D),
        in_specs=[blk5, blk5],
        out_specs=[blk5, blk4],
        out_shape=[
            jax.ShapeDtypeStruct((N, D, 3, H, W), jnp.float32),
            jax.ShapeDtypeStruct((N, D, H, W), jnp.int32),
        ],
    )(flw5, grd5)


def _sc_build(img):
    mesh = plsc.VectorSubcoreMesh(core_axis_name="c", subcore_axis_name="s")

    @functools.partial(
        pl.kernel,
        out_type=jax.ShapeDtypeStruct((NP, 8), jnp.float32),
        mesh=mesh,
        compiler_params=_CPARAMS,
        scratch_types=[
            pltpu.VMEM((2, 2, HW), jnp.float32),
            pltpu.VMEM((2, 2048, 8), jnp.float32),
            pltpu.SemaphoreType.DMA,
            pltpu.SemaphoreType.DMA,
        ],
    )
    def build(img_h, tab_h, pbuf, obuf, psem, osem):
        wid = lax.axis_index("s") * 2 + lax.axis_index("c")
        lane = lax.iota(jnp.int32, 16)
        cv0 = lane * 0

        def plane_copies(t, slot):
            q = wid * TPT + t
            zoff = pl.multiple_of(q * HW, HW)
            z = q & (D - 1)
            zp_off = pl.multiple_of(
                jnp.where(z == D - 1, zoff, zoff + HW), HW)
            yield (img_h.at[pl.ds(zoff, HW)], pbuf.at[slot, 0])
            yield (img_h.at[pl.ds(zp_off, HW)], pbuf.at[slot, 1])

        for s, d in plane_copies(0, 0):
            pltpu.async_copy(s, d, psem)

        @pl.loop(0, TPT)
        def _task(t):
            q = wid * TPT + t
            slot = t & 1

            @pl.when(t + 1 < TPT)
            def _():
                for s, d in plane_copies(t + 1, 1 - (t & 1)):
                    pltpu.async_copy(s, d, psem)

            for s, d in plane_copies(t, slot):
                pltpu.make_async_copy(s, d, psem).wait()
            pbuf0 = pbuf.at[slot, 0]
            pbuf1 = pbuf.at[slot, 1]

            @pl.loop(0, 8)
            def _ychunk(yc):
                buf = yc & 1
                dst = pl.multiple_of(q * HW + yc * 2048, 2048)

                @pl.when(yc >= 2)
                def _():
                    pltpu.make_async_copy(
                        obuf.at[buf], tab_h.at[pl.ds(dst, 2048)],
                        osem).wait()

                @pl.loop(0, 16)
                def _line(l):
                    y = yc * 16 + l
                    ro0 = y * W
                    ro1 = jnp.minimum(y + 1, H - 1) * W
                    ob = obuf.at[buf]

                    @pl.loop(0, 8, unroll=2)
                    def _xg(xg):
                        xo = xg * 16
                        xe = xo + lane
                        xc = jnp.minimum(xe + 1, W - 1)
                        c = 0
                        for pb in (pbuf0, pbuf1):
                            for ro in (ro0, ro1):
                                ve = pb[pl.ds(pl.multiple_of(ro + xo, 16),
                                              16)]
                                vo = plsc.load_gather(pb, [ro + xc])
                                pt = l * W + xe
                                plsc.store_scatter(ob, [pt, cv0 + c], ve)
                                plsc.store_scatter(ob, [pt, cv0 + c + 1], vo)
                                c += 2

                pltpu.async_copy(obuf.at[buf],
                                 tab_h.at[pl.ds(dst, 2048)], osem)

            @pl.loop(0, 2)
            def _tail(i):
                pltpu.make_async_copy(
                    obuf.at[i], tab_h.at[pl.ds(0, 2048)],
                    osem).wait()

    return build(img)


def _sc_warp(tab, base, frac):
    mesh = plsc.VectorSubcoreMesh(core_axis_name="c", subcore_axis_name="s")

    @functools.partial(
        pl.kernel,
        out_type=jax.ShapeDtypeStruct((NP,), jnp.float32),
        mesh=mesh,
        compiler_params=_CPARAMS,
        scratch_types=[
            pltpu.VMEM((2, 3 * K), jnp.float32),
            pltpu.VMEM((2, K), jnp.int32),
            pltpu.VMEM((2, K, 8), jnp.float32),
            pltpu.VMEM((2, K), jnp.float32),
            pltpu.SemaphoreType.DMA,
            pltpu.SemaphoreType.DMA,
            pltpu.SemaphoreType.DMA,
        ],
    )
    def warp(tab_h, base_h, frac_h, out_h, frbuf, idxbuf, vbuf, outbuf,
             isem, gsem, osem):
        wid = lax.axis_index("s") * 2 + lax.axis_index("c")
        lane = lax.iota(jnp.int32, 16)
        cvecs = [lane * 0 + c for c in range(8)]

        def in_copies(ch, slot):
            p0 = pl.multiple_of(wid * PER_TILE + ch * K, K)
            nz = p0 // HW
            yo = p0 - nz * HW
            yield (base_h.at[pl.ds(p0, K)], idxbuf.at[slot])
            for c in range(3):
                src = pl.multiple_of((nz * 3 + c) * HW + yo, K)
                yield (frac_h.at[pl.ds(src, K)],
                       frbuf.at[slot, pl.ds(c * K, K)])

        def fire_in(ch, slot):
            for s, d in in_copies(ch, slot):
                pltpu.async_copy(s, d, isem)

        def wait_in(ch, slot):
            for s, d in in_copies(ch, slot):
                pltpu.make_async_copy(s, d, isem).wait()

        def gather_copies(slot):
            yield (tab_h.at[idxbuf.at[slot]], vbuf.at[slot])

        def blend(ch, slot):
            @pl.loop(0, K // 16, unroll=4)
            def _acc(j):
                o = pl.multiple_of(j * 16, 16)
                wx = frbuf[slot, pl.ds(0 * K + o, 16)]
                wy = frbuf[slot, pl.ds(1 * K + o, 16)]
                wz = frbuf[slot, pl.ds(2 * K + o, 16)]
                ux = 1.0 - wx
                uy = 1.0 - wy
                uz = 1.0 - wz
                rows = o + lane
                vb = vbuf.at[slot]
                v = [plsc.load_gather(vb, [rows, cvecs[c]])
                     for c in range(8)]
                m0 = v[0] * ux + v[1] * wx
                m1 = v[2] * ux + v[3] * wx
                m2 = v[4] * ux + v[5] * wx
                m3 = v[6] * ux + v[7] * wx
                acc = (m0 * uy + m1 * wy) * uz + (m2 * uy + m3 * wy) * wz
                outbuf[slot, pl.ds(o, 16)] = acc

        def out_copy(ch, slot):
            p0 = pl.multiple_of(wid * PER_TILE + ch * K, K)
            return (outbuf.at[slot], out_h.at[pl.ds(p0, K)])

        fire_in(0, 0)

        @pl.loop(0, NCH)
        def _chunk(ch):
            slot = ch & 1
            pslot = 1 - slot

            wait_in(ch, slot)

            @pl.when(ch >= 1)
            def _():
                for s, d in gather_copies(pslot):
                    pltpu.make_async_copy(s, d, gsem).wait()

            for s, d in gather_copies(slot):
                pltpu.async_copy(s, d, gsem)

            @pl.when(ch >= 1)
            def _():
                @pl.when(ch >= 3)
                def _():
                    s, d = out_copy(ch - 3, pslot)
                    pltpu.make_async_copy(s, d, osem).wait()

                blend(ch - 1, pslot)
                s, d = out_copy(ch - 1, pslot)
                pltpu.async_copy(s, d, osem)

            @pl.when(ch + 1 < NCH)
            def _():
                fire_in(ch + 1, pslot)

        last = NCH - 1
        lslot = last & 1
        for s, d in gather_copies(lslot):
            pltpu.make_async_copy(s, d, gsem).wait()
        s, d = out_copy(last - 2, lslot)
        pltpu.make_async_copy(s, d, osem).wait()
        blend(last, lslot)
        s, d = out_copy(last - 1, 1 - lslot)
        pltpu.make_async_copy(s, d, osem).wait()
        s, d = out_copy(last, lslot)
        pltpu.sync_copy(s, d)

    return warp(tab, base, frac)


def kernel(input_image, flow, grid):
    assert input_image.shape == (N, C, D, H, W)
    ft5 = flow.transpose(0, 1, 4, 2, 3)
    gt5 = grid.transpose(0, 1, 4, 2, 3)
    frac, base = _tc_coords(ft5, gt5)
    tab = _sc_build(input_image.reshape(-1))
    out = _sc_warp(tab, base.reshape(-1), frac.reshape(-1))
    return out.reshape(input_image.shape)
